# Initial kernel scaffold; baseline (speedup 1.0000x reference)
import jax
import jax.numpy as jnp
from jax.experimental import pallas as pl


def kernel(x, edge_index, W1, b1, W2, b2, W3, b3):
    raise NotImplementedError("write your pallas kernel here")


# speedup vs baseline: 12.4421x; 12.4421x over previous
import functools

import jax
import jax.numpy as jnp
from jax import lax
from jax.experimental import pallas as pl
from jax.experimental.pallas import tpu as pltpu
import jax.experimental.pallas.tpu_sc as plsc

_N = 10000
_E = 320000
_D = 128
_NCLS = 40
_NC = 2
_NS = 16
_NW = _NC * _NS
_EPW = _E // _NW
_C = 80
_NCHUNK = _EPW // _C
_STR = 640
_NPAD = _NS * _STR

_ZCH = _STR // _C

_sc_mesh = plsc.VectorSubcoreMesh(core_axis_name="c", subcore_axis_name="s")


def _deg_body(dst_hbm, out_hbm, dstv, onesv, zbuf, dacc):
    c = lax.axis_index("c")
    s = lax.axis_index("s")
    wid = s * _NC + c

    def fill_z(j, carry):
        zbuf[pl.ds(j * 16, 16)] = jnp.zeros((16,), jnp.float32)
        return carry

    lax.fori_loop(0, _STR // 16, fill_z, 0)

    def fill_o(j, carry):
        onesv[pl.ds(j * 16, 16)] = jnp.ones((16,), jnp.float32)
        return carry

    lax.fori_loop(0, _C // 16, fill_o, 0)

    pltpu.sync_copy(zbuf, dacc.at[pl.ds(s * _STR, _STR)])
    plsc.subcore_barrier()

    base0 = wid * _EPW

    def body(i, carry):
        pltpu.sync_copy(dst_hbm.at[pl.ds(base0 + i * _C, _C)], dstv)
        pltpu.sync_copy(onesv, dacc.at[dstv], add=True)
        return carry

    lax.fori_loop(0, _NCHUNK, body, 0)
    plsc.subcore_barrier()

    @pl.when(s == 0)
    def _():
        pltpu.sync_copy(dacc, out_hbm.at[c])


_deg_call = pl.kernel(
    _deg_body,
    out_type=jax.ShapeDtypeStruct((_NC, _NPAD), jnp.float32),
    mesh=_sc_mesh,
    scratch_types=[
        pltpu.VMEM((_C,), jnp.int32),
        pltpu.VMEM((_C,), jnp.float32),
        pltpu.VMEM((_STR,), jnp.float32),
        pltpu.VMEM_SHARED((_NPAD,), jnp.float32),
    ],
)


def _gather_scatter_body(g_hbm, src_hbm, dst_hbm, out_hbm,
                         srcv, dstv, rows, acc, sem):
    c = lax.axis_index("c")
    s = lax.axis_index("s")
    wid = s * _NC + c

    def fz(i, carry):
        def fz2(j, carry2):
            rows[i, pl.ds(j * 16, 16)] = jnp.zeros((16,), jnp.float32)
            return carry2

        lax.fori_loop(0, _D // 16, fz2, 0)
        return carry

    lax.fori_loop(0, _C, fz, 0)

    def za(k, carry):
        pltpu.sync_copy(rows, acc.at[pl.ds(s * _STR + k * _C, _C)])
        return carry

    lax.fori_loop(0, _ZCH, za, 0)
    plsc.subcore_barrier()

    base0 = wid * _EPW

    def body(i, carry):
        base = base0 + i * _C
        pltpu.sync_copy(src_hbm.at[pl.ds(base, _C)], srcv)
        pltpu.sync_copy(dst_hbm.at[pl.ds(base, _C)], dstv)
        pltpu.async_copy(g_hbm.at[srcv], rows, sem).wait()
        pltpu.sync_copy(rows, acc.at[dstv], add=True)
        return carry

    lax.fori_loop(0, _NCHUNK, body, 0)
    plsc.subcore_barrier()

    pltpu.sync_copy(acc.at[pl.ds(s * _STR, _STR)],
                    out_hbm.at[c, pl.ds(s * _STR, _STR)])


_gather_scatter_call = pl.kernel(
    _gather_scatter_body,
    out_type=jax.ShapeDtypeStruct((_NC, _NPAD, _D), jnp.float32),
    mesh=_sc_mesh,
    scratch_types=[
        pltpu.VMEM((_C,), jnp.int32),
        pltpu.VMEM((_C,), jnp.int32),
        pltpu.VMEM((_C, _D), jnp.float32),
        pltpu.VMEM_SHARED((_NPAD, _D), jnp.float32),
        pltpu.SemaphoreType.DMA,
    ],
)


_R = 400
_G = _N // _R


def _tc_first_body(x_ref, w_ref, d0_ref, d1_ref, h_ref, g_ref, di_ref):
    deg = d0_ref[...] + d1_ref[...] + 1.0
    dinv = lax.rsqrt(deg)
    h = jnp.dot(x_ref[...], w_ref[...], preferred_element_type=jnp.float32)
    h_ref[...] = h
    g_ref[...] = h * dinv
    di_ref[...] = dinv


_tc_first = pl.pallas_call(
    _tc_first_body,
    grid=(_G,),
    in_specs=[
        pl.BlockSpec((_R, _D), lambda i: (i, 0)),
        pl.BlockSpec((_D, _D), lambda i: (0, 0)),
        pl.BlockSpec((_R, 1), lambda i: (i, 0)),
        pl.BlockSpec((_R, 1), lambda i: (i, 0)),
    ],
    out_specs=[
        pl.BlockSpec((_R, _D), lambda i: (i, 0)),
        pl.BlockSpec((_R, _D), lambda i: (i, 0)),
        pl.BlockSpec((_R, 1), lambda i: (i, 0)),
    ],
    out_shape=[
        jax.ShapeDtypeStruct((_N, _D), jnp.float32),
        jax.ShapeDtypeStruct((_N, _D), jnp.float32),
        jax.ShapeDtypeStruct((_N, 1), jnp.float32),
    ],
)


def _tc_mid_body(s0_ref, s1_ref, h_ref, di_ref, b_ref, w_ref,
                 h2_ref, g2_ref):
    di = di_ref[...]
    t = di * (s0_ref[...] + s1_ref[...]) + di * di * h_ref[...] + b_ref[...]
    t = jnp.maximum(t, 0.0)
    h2 = jnp.dot(t, w_ref[...], preferred_element_type=jnp.float32)
    h2_ref[...] = h2
    g2_ref[...] = h2 * di


_tc_mid = pl.pallas_call(
    _tc_mid_body,
    grid=(_G,),
    in_specs=[
        pl.BlockSpec((_R, _D), lambda i: (i, 0)),
        pl.BlockSpec((_R, _D), lambda i: (i, 0)),
        pl.BlockSpec((_R, _D), lambda i: (i, 0)),
        pl.BlockSpec((_R, 1), lambda i: (i, 0)),
        pl.BlockSpec((1, _D), lambda i: (0, 0)),
        pl.BlockSpec((_D, _D), lambda i: (0, 0)),
    ],
    out_specs=[
        pl.BlockSpec((_R, _D), lambda i: (i, 0)),
        pl.BlockSpec((_R, _D), lambda i: (i, 0)),
    ],
    out_shape=[
        jax.ShapeDtypeStruct((_N, _D), jnp.float32),
        jax.ShapeDtypeStruct((_N, _D), jnp.float32),
    ],
)


def _tc_last_body(s0_ref, s1_ref, h_ref, di_ref, b_ref, w_ref, b3_ref,
                  out_ref):
    di = di_ref[...]
    t = di * (s0_ref[...] + s1_ref[...]) + di * di * h_ref[...] + b_ref[...]
    t = jnp.maximum(t, 0.0)
    out_ref[...] = (
        jnp.dot(t, w_ref[...], preferred_element_type=jnp.float32)
        + b3_ref[...]
    )


_tc_last = pl.pallas_call(
    _tc_last_body,
    grid=(_G,),
    in_specs=[
        pl.BlockSpec((_R, _D), lambda i: (i, 0)),
        pl.BlockSpec((_R, _D), lambda i: (i, 0)),
        pl.BlockSpec((_R, _D), lambda i: (i, 0)),
        pl.BlockSpec((_R, 1), lambda i: (i, 0)),
        pl.BlockSpec((1, _D), lambda i: (0, 0)),
        pl.BlockSpec((_D, _NCLS), lambda i: (0, 0)),
        pl.BlockSpec((1, _NCLS), lambda i: (0, 0)),
    ],
    out_specs=pl.BlockSpec((_R, _NCLS), lambda i: (i, 0)),
    out_shape=jax.ShapeDtypeStruct((_N, _NCLS), jnp.float32),
)


def kernel(x, edge_index, W1, b1, W2, b2, W3, b3):
    src = edge_index[0]
    dst = edge_index[1]

    deg_p = _deg_call(dst)
    dp0 = deg_p[0, :_N, None]
    dp1 = deg_p[1, :_N, None]

    h1, g1, dinv = _tc_first(x, W1, dp0, dp1)

    acc1 = _gather_scatter_call(g1, src, dst)
    h2, g2 = _tc_mid(acc1[0, :_N], acc1[1, :_N], h1, dinv,
                     b1.reshape(1, _D), W2)

    acc2 = _gather_scatter_call(g2, src, dst)
    return _tc_last(acc2[0, :_N], acc2[1, :_N], h2, dinv,
                    b2.reshape(1, _D), W3, b3.reshape(1, _NCLS))
